# Initial kernel scaffold; baseline (speedup 1.0000x reference)
import jax
import jax.numpy as jnp
from jax.experimental import pallas as pl


def kernel(x, edge_index, batch, params):
    raise NotImplementedError("write your pallas kernel here")


# speedup vs baseline: 1.5360x; 1.5360x over previous
import functools

import jax
import jax.numpy as jnp
from jax import lax
from jax.experimental import pallas as pl
from jax.experimental.pallas import tpu as pltpu
from jax.experimental.pallas import tpu_sc as plsc

NC, NS = 2, 16
NW = NC * NS
F = 128
EPS = 1e-5
NUM_GRAPHS = 64


def _sc_mesh():
    return plsc.VectorSubcoreMesh(core_axis_name="c", subcore_axis_name="s",
                                  num_cores=NC, num_subcores=NS)


@functools.partial(jax.jit, static_argnames=("n_nodes", "n_edges"))
def _sc_gather(P, Q, dsts, srcs, *, n_nodes, n_edges):
    per_w = n_edges // NW
    CH = 128
    nch = per_w // CH
    tail = per_w - nch * CH

    @functools.partial(
        pl.kernel, mesh=_sc_mesh(),
        out_type=(jax.ShapeDtypeStruct((n_edges, F), jnp.float32),
                  jax.ShapeDtypeStruct((n_edges, F), jnp.float32)),
        scratch_types=[
            pltpu.VMEM((CH,), jnp.int32),
            pltpu.VMEM((CH,), jnp.int32),
            pltpu.VMEM((CH, F), jnp.float32),
            pltpu.VMEM((CH, F), jnp.float32),
            pltpu.VMEM((16,), jnp.int32),
            pltpu.VMEM((16,), jnp.int32),
            pltpu.VMEM((16, F), jnp.float32),
            pltpu.VMEM((16, F), jnp.float32),
            pltpu.SemaphoreType.DMA,
        ],
    )
    def gather_k(p_hbm, q_hbm, d_hbm, s_hbm, pg_hbm, qg_hbm,
                 idxd, idxs, rowsp, rowsq, idxdt, idxst, rowspt, rowsqt, sem):
        wid = lax.axis_index("s") * NC + lax.axis_index("c")
        base = wid * per_w

        def chunk(c, _):
            off = pl.multiple_of(base + c * CH, 8)
            pltpu.sync_copy(d_hbm.at[pl.ds(off, CH)], idxd)
            pltpu.sync_copy(s_hbm.at[pl.ds(off, CH)], idxs)
            cp = pltpu.async_copy(p_hbm.at[idxd], rowsp, sem)
            cq = pltpu.async_copy(q_hbm.at[idxs], rowsq, sem)
            cp.wait()
            cq.wait()
            pltpu.sync_copy(rowsp, pg_hbm.at[pl.ds(off, CH)])
            pltpu.sync_copy(rowsq, qg_hbm.at[pl.ds(off, CH)])
            return 0

        lax.fori_loop(0, nch, chunk, 0)
        if tail:
            off = pl.multiple_of(base + nch * CH, 8)
            pltpu.sync_copy(d_hbm.at[pl.ds(off, tail)], idxdt)
            pltpu.sync_copy(s_hbm.at[pl.ds(off, tail)], idxst)
            cp = pltpu.async_copy(p_hbm.at[idxdt], rowspt, sem)
            cq = pltpu.async_copy(q_hbm.at[idxst], rowsqt, sem)
            cp.wait()
            cq.wait()
            pltpu.sync_copy(rowspt, pg_hbm.at[pl.ds(off, tail)])
            pltpu.sync_copy(rowsqt, qg_hbm.at[pl.ds(off, tail)])

    return gather_k(P, Q, dsts, srcs)


@functools.partial(jax.jit, static_argnames=("n_pad", "e_pad"))
def _sc_segmax(z_flat, dst_pad, ws_pad, *, n_pad, e_pad):
    per_n = n_pad // NW
    CH = 128
    acc_w = (per_n + 1) * F

    @functools.partial(
        pl.kernel, mesh=_sc_mesh(),
        out_type=jax.ShapeDtypeStruct((n_pad * F,), jnp.float32),
        scratch_types=[
            pltpu.VMEM((acc_w,), jnp.float32),
            pltpu.VMEM((CH * F,), jnp.float32),
            pltpu.VMEM((CH,), jnp.int32),
            pltpu.VMEM((NW * 16,), jnp.int32),
        ],
    )
    def segmax_k(z_hbm, d_hbm, ws_hbm, agg_hbm, acc, zbuf, dbuf, wsv):
        wid = lax.axis_index("s") * NC + lax.axis_index("c")
        n0 = wid * per_n
        neg = jnp.full((16,), -jnp.inf, jnp.float32)

        def initb(i, _):
            acc[pl.ds(i * 16, 16)] = neg
            return 0

        lax.fori_loop(0, acc_w // 16, initb, 0)

        pltpu.sync_copy(ws_hbm, wsv)
        wsrow = wsv[pl.ds(wid * 16, 16)]
        e_lo = wsrow[0]
        e_hi = wsrow[1]
        e_lo_al = (e_lo // 8) * 8
        nch = (e_hi - e_lo_al + CH - 1) // CH

        def chunk(c, _):
            off = pl.multiple_of(e_lo_al + c * CH, 8)
            pltpu.sync_copy(d_hbm.at[pl.ds(off, CH)], dbuf)
            pltpu.sync_copy(z_hbm.at[pl.ds(off * F, CH * F)], zbuf)

            def grp16(g, _):
                dv = dbuf[pl.ds(g * 16, 16)]
                for j in range(16):
                    d = dv[j]
                    inr = jnp.logical_and(d >= n0, d < n0 + per_n)
                    ro = jnp.where(inr, d - n0, per_n) * F
                    zo = (g * 16 + j) * F
                    for k in range(F // 16):
                        a = acc[pl.ds(ro + k * 16, 16)]
                        zv = zbuf[pl.ds(zo + k * 16, 16)]
                        acc[pl.ds(ro + k * 16, 16)] = jnp.maximum(a, zv)
                return 0

            lax.fori_loop(0, CH // 16, grp16, 0)
            return 0

        lax.fori_loop(0, nch, chunk, 0)
        out_off = pl.multiple_of(n0 * F, 8)
        pltpu.sync_copy(acc.at[pl.ds(0, per_n * F)],
                        agg_hbm.at[pl.ds(out_off, per_n * F)])

    return segmax_k(z_flat, dst_pad, ws_pad)


def _msg_call(xi, xj, w1, b1, e_pad, blk, nblk):
    def body(xi_ref, xj_ref, w1_ref, b1_ref, m_ref, st_ref):
        @pl.when(pl.program_id(0) == 0)
        def _():
            st_ref[...] = jnp.zeros_like(st_ref)

        xi_v = xi_ref[...]
        cc = jnp.concatenate([xi_v, xj_ref[...] - xi_v], axis=1)
        m = jnp.dot(cc, w1_ref[...],
                    preferred_element_type=jnp.float32) + b1_ref[...]
        m_ref[...] = m
        st_ref[0:1, :] = st_ref[0:1, :] + jnp.sum(m, 0, keepdims=True)
        st_ref[1:2, :] = st_ref[1:2, :] + jnp.sum(m * m, 0, keepdims=True)

    return pl.pallas_call(
        body,
        grid=(nblk,),
        in_specs=[pl.BlockSpec((blk, F), lambda i: (i, 0)),
                  pl.BlockSpec((blk, F), lambda i: (i, 0)),
                  pl.BlockSpec((2 * F, F), lambda i: (0, 0)),
                  pl.BlockSpec((1, F), lambda i: (0, 0))],
        out_specs=(pl.BlockSpec((blk, F), lambda i: (i, 0)),
                   pl.BlockSpec((8, F), lambda i: (0, 0))),
        out_shape=(jax.ShapeDtypeStruct((e_pad, F), jnp.float32),
                   jax.ShapeDtypeStruct((8, F), jnp.float32)),
    )(xi, xj, w1, b1)


def _edge_call(m, st, pv, w2, n_edges, e_pad, blk, nblk):
    inv_e = 1.0 / n_edges

    def body(m_ref, st_ref, pv_ref, w2_ref, z_ref):
        mu = st_ref[0:1, :] * inv_e
        var = st_ref[1:2, :] * inv_e - mu * mu
        t = ((m_ref[...] - mu) / jnp.sqrt(var + EPS) * pv_ref[0:1, :]
             + pv_ref[1:2, :])
        t = jnp.maximum(t, 0.0)
        z_ref[...] = jnp.dot(t, w2_ref[...],
                             preferred_element_type=jnp.float32) + pv_ref[2:3, :]

    return pl.pallas_call(
        body,
        grid=(nblk,),
        in_specs=[pl.BlockSpec((blk, F), lambda i: (i, 0)),
                  pl.BlockSpec((8, F), lambda i: (0, 0)),
                  pl.BlockSpec((8, F), lambda i: (0, 0)),
                  pl.BlockSpec((F, F), lambda i: (0, 0))],
        out_specs=pl.BlockSpec((blk, F), lambda i: (i, 0)),
        out_shape=jax.ShapeDtypeStruct((e_pad, F), jnp.float32),
    )(m, st, pv, w2)


def _node_call(agg, gb, n):
    inv_n = 1.0 / n

    def body(a_ref, gb_ref, h_ref):
        a = a_ref[...]
        a = jnp.where(jnp.isfinite(a), a, 0.0)
        m = jnp.sum(a, 0, keepdims=True) * inv_n
        v = jnp.sum(a * a, 0, keepdims=True) * inv_n - m * m
        hn = (a - m) / jnp.sqrt(v + EPS) * gb_ref[0:1, :] + gb_ref[1:2, :]
        h_ref[...] = jnp.maximum(hn, 0.0)

    return pl.pallas_call(
        body,
        grid=(1,),
        in_specs=[pl.BlockSpec((n, F), lambda i: (0, 0)),
                  pl.BlockSpec((8, F), lambda i: (0, 0))],
        out_specs=pl.BlockSpec((n, F), lambda i: (0, 0)),
        out_shape=jax.ShapeDtypeStruct((n, F), jnp.float32),
    )(agg, gb)


def _pool_call(h1, h2, h3, oh, wj1, wj2, wj3, bjk, n, blk, nblk):
    def body(h1_ref, h2_ref, h3_ref, oh_ref, w1_ref, w2_ref, w3_ref, bj_ref,
             g_ref):
        @pl.when(pl.program_id(0) == 0)
        def _():
            g_ref[...] = jnp.zeros_like(g_ref)

        hb = (jnp.dot(h1_ref[...], w1_ref[...],
                      preferred_element_type=jnp.float32)
              + jnp.dot(h2_ref[...], w2_ref[...],
                        preferred_element_type=jnp.float32)
              + jnp.dot(h3_ref[...], w3_ref[...],
                        preferred_element_type=jnp.float32)
              + bj_ref[...])
        g_ref[...] = g_ref[...] + lax.dot_general(
            oh_ref[...], hb, (((0,), (0,)), ((), ())),
            preferred_element_type=jnp.float32, precision=lax.Precision.HIGHEST)

    return pl.pallas_call(
        body,
        grid=(nblk,),
        in_specs=[pl.BlockSpec((blk, F), lambda i: (i, 0)),
                  pl.BlockSpec((blk, F), lambda i: (i, 0)),
                  pl.BlockSpec((blk, F), lambda i: (i, 0)),
                  pl.BlockSpec((blk, NUM_GRAPHS), lambda i: (i, 0)),
                  pl.BlockSpec((F, F), lambda i: (0, 0)),
                  pl.BlockSpec((F, F), lambda i: (0, 0)),
                  pl.BlockSpec((F, F), lambda i: (0, 0)),
                  pl.BlockSpec((1, F), lambda i: (0, 0))],
        out_specs=pl.BlockSpec((NUM_GRAPHS, F), lambda i: (0, 0)),
        out_shape=jax.ShapeDtypeStruct((NUM_GRAPHS, F), jnp.float32),
    )(h1, h2, h3, oh, wj1, wj2, wj3, bjk)


def _cls_call(g, wc1, bc1, gc, bc, wc2, bc2):
    ng = g.shape[0]
    h2 = wc1.shape[1]
    nc = wc2.shape[1]
    inv = 1.0 / ng

    def body(g_ref, w1_ref, b1_ref, gc_ref, bc_ref, w2_ref, b2_ref, o_ref):
        a = jnp.dot(g_ref[...], w1_ref[...],
                    preferred_element_type=jnp.float32) + b1_ref[...]
        m = jnp.sum(a, 0, keepdims=True) * inv
        v = jnp.sum(a * a, 0, keepdims=True) * inv - m * m
        a = (a - m) / jnp.sqrt(v + EPS) * gc_ref[...] + bc_ref[...]
        a = jnp.maximum(a, 0.0)
        o_ref[...] = jnp.dot(a, w2_ref[...],
                             preferred_element_type=jnp.float32) + b2_ref[...]

    return pl.pallas_call(
        body,
        out_shape=jax.ShapeDtypeStruct((ng, nc), jnp.float32),
    )(g, wc1, bc1, gc, bc, wc2, bc2)


def kernel(x, edge_index, batch, params):
    n, f = x.shape
    e = edge_index.shape[1]
    assert f == F and e % NW == 0

    per_n = -(-n // NW)
    n_pad = per_n * NW
    e_pad = e + 128

    src = edge_index[0]
    dst = edge_index[1]
    dst_s, src_s = lax.sort((dst, src), num_keys=1)
    bounds = jnp.arange(NW + 1, dtype=jnp.int32) * per_n
    ws = jnp.searchsorted(dst_s, bounds, side="left").astype(jnp.int32)
    ws_pad = (jnp.zeros((NW, 16), jnp.int32)
              .at[:, 0].set(ws[:-1]).at[:, 1].set(ws[1:]).reshape(-1))
    dst_pad = jnp.concatenate(
        [dst_s, jnp.full((e_pad - e,), n_pad, jnp.int32)])
    oh = jax.nn.one_hot(batch, NUM_GRAPHS, dtype=jnp.float32)

    blk_e, nblk_e = 2000, e // 2000
    blk_n, nblk_n = 2000, n // 2000

    convs = params["convs"]
    h = x
    hs = []
    for l in range(len(convs)):
        c = convs[l]
        pv = jnp.stack([c["g1"], c["be1"], c["b2"],
                        jnp.zeros(F), jnp.zeros(F), jnp.zeros(F),
                        jnp.zeros(F), jnp.zeros(F)]).astype(jnp.float32)
        gb = jnp.stack([c["gn"], c["bn"],
                        jnp.zeros(F), jnp.zeros(F), jnp.zeros(F),
                        jnp.zeros(F), jnp.zeros(F), jnp.zeros(F)]).astype(
                            jnp.float32)
        Xi, Xj = _sc_gather(h, h, dst_s, src_s, n_nodes=n, n_edges=e)
        m, st = _msg_call(Xi, Xj, c["W1"], c["b1"].reshape(1, F),
                          e_pad, blk_e, nblk_e)
        z = _edge_call(m, st, pv, c["W2"], e, e_pad, blk_e, nblk_e)
        agg_f = _sc_segmax(z.reshape(-1), dst_pad, ws_pad,
                           n_pad=n_pad, e_pad=e_pad)
        agg = agg_f.reshape(n_pad, F)
        h = _node_call(agg, gb, n)
        hs.append(h)

    wjk = params["Wjk"]
    G = _pool_call(hs[0], hs[1], hs[2], oh,
                   wjk[0:F], wjk[F:2 * F], wjk[2 * F:3 * F],
                   params["bjk"].reshape(1, F), n, blk_n, nblk_n)
    out = _cls_call(G, params["Wc1"], params["bc1"].reshape(1, -1),
                    params["gc"].reshape(1, -1), params["bc"].reshape(1, -1),
                    params["Wc2"], params["bc2"].reshape(1, -1))
    return out
